# baseline (speedup 1.0000x reference)
import functools

import jax
import jax.numpy as jnp
from jax.experimental import pallas as pl
from jax.experimental.pallas import tpu as pltpu
from jax.experimental.pallas import tpu_sc as plsc

B = 4
S = 8192
D = 768
N = B * S
C = 128
NC = D // C
W = 128
NWIN = N // W
LANES = 16


def _pe_add_sc(x2d, fidx, table_flat):
    mesh = plsc.VectorSubcoreMesh(core_axis_name="c", subcore_axis_name="s")

    @functools.partial(
        pl.kernel,
        out_type=jax.ShapeDtypeStruct((N, D), jnp.float32),
        mesh=mesh,
    )
    def k(x_hbm, i_hbm, t_hbm, o_hbm):
        def body(i_vmem, x_vmem, o_vmem):
            pltpu.sync_copy(t_hbm.at[i_vmem.at[0]], o_vmem)

            for r in range(W):
                for c in range(0, C, LANES):
                    slc = (pl.ds(r, 1), pl.ds(c, LANES))
                    plsc.addupdate(o_vmem.at[slc], x_vmem.at[slc][...])

        pltpu.emit_pipeline(
            body,
            grid=(NWIN, NC),
            in_specs=[
                pl.BlockSpec((1, W), lambda i, j: (i * NC + j, 0)),
                pl.BlockSpec((W, C), lambda i, j: (i, j)),
            ],
            out_specs=[pl.BlockSpec((W, C), lambda i, j: (i, j))],
            core_axis_name=("c", "s"),
            dimension_semantics=(pltpu.PARALLEL, pltpu.PARALLEL),
        )(i_hbm, x_hbm, o_hbm)

    return k(x2d, fidx, table_flat)


def kernel(x, position, pos_table):
    x2d = x.reshape(N, D)
    pos = position.reshape(NWIN, W).astype(jnp.int32)
    fidx = (pos[:, None, :] * NC + jnp.arange(NC, dtype=jnp.int32)[None, :, None])
    fidx = fidx.reshape(NWIN * NC, W)
    table_flat = pos_table.reshape(8192 * NC, C)
    out = _pe_add_sc(x2d, fidx, table_flat)
    return out.reshape(B, S, D)


# speedup vs baseline: 1.8943x; 1.8943x over previous
import functools

import jax
import jax.numpy as jnp
from jax.experimental import pallas as pl
from jax.experimental.pallas import tpu as pltpu
from jax.experimental.pallas import tpu_sc as plsc

B = 4
S = 8192
D = 768
N = B * S
C = 128
NC = D // C
W = 128
NWIN = N // W

K = 4
NK = N // K
NWK = NWIN // K

TC_ROWS = 2048
TB = NK // TC_ROWS


def _gather_sc(fidx, table_flat):
    mesh = plsc.VectorSubcoreMesh(core_axis_name="c", subcore_axis_name="s")

    @functools.partial(
        pl.kernel,
        out_type=jax.ShapeDtypeStruct((NK, D), jnp.float32),
        mesh=mesh,
    )
    def k(i_hbm, t_hbm, o_hbm):
        def body(i_vmem, o_vmem):
            pltpu.sync_copy(t_hbm.at[i_vmem.at[0]], o_vmem)

        pltpu.emit_pipeline(
            body,
            grid=(NWK, NC),
            in_specs=[pl.BlockSpec((1, W), lambda i, j: (i * NC + j, 0))],
            out_specs=[pl.BlockSpec((W, C), lambda i, j: (i, j))],
            core_axis_name=("c", "s"),
            dimension_semantics=(pltpu.PARALLEL, pltpu.PARALLEL),
        )(i_hbm, o_hbm)

    return k(fidx, table_flat)


def _add_first(x2d, pe0):
    def body(x_ref, pe_ref, o_ref):
        o_ref[...] = x_ref[...] + pe_ref[...]

    return pl.pallas_call(
        body,
        out_shape=jax.ShapeDtypeStruct((N, D), jnp.float32),
        grid=(TB,),
        in_specs=[
            pl.BlockSpec((TC_ROWS, D), lambda i: (i, 0)),
            pl.BlockSpec((TC_ROWS, D), lambda i: (i, 0)),
        ],
        out_specs=pl.BlockSpec((TC_ROWS, D), lambda i: (i, 0)),
    )(x2d, pe0)


def _add_chunk(prev, x2d, pe, chunk):
    def body(prev_ref, x_ref, pe_ref, o_ref):
        o_ref[...] = x_ref[...] + pe_ref[...]

    off = chunk * TB
    return pl.pallas_call(
        body,
        out_shape=jax.ShapeDtypeStruct((N, D), jnp.float32),
        grid=(TB,),
        in_specs=[
            pl.BlockSpec(memory_space=pltpu.MemorySpace.HBM),
            pl.BlockSpec((TC_ROWS, D), lambda i: (i + off, 0)),
            pl.BlockSpec((TC_ROWS, D), lambda i: (i, 0)),
        ],
        out_specs=pl.BlockSpec((TC_ROWS, D), lambda i: (i + off, 0)),
        input_output_aliases={0: 0},
    )(prev, x2d, pe)


def kernel(x, position, pos_table):
    x2d = x.reshape(N, D)
    pos = position.reshape(NWIN, W).astype(jnp.int32)
    fidx = (pos[:, None, :] * NC + jnp.arange(NC, dtype=jnp.int32)[None, :, None])
    fidx = fidx.reshape(K, NWK * NC, W)
    table_flat = pos_table.reshape(8192 * NC, C)
    pe = [_gather_sc(fidx[k], table_flat) for k in range(K)]
    out = _add_first(x2d, pe[0])
    for k in range(1, K):
        out = _add_chunk(out, x2d, pe[k], k)
    return out.reshape(B, S, D)


# speedup vs baseline: 2.2269x; 1.1755x over previous
import functools

import jax
import jax.numpy as jnp
from jax.experimental import pallas as pl
from jax.experimental.pallas import tpu as pltpu
from jax.experimental.pallas import tpu_sc as plsc

B = 4
S = 8192
D = 768
N = B * S
C = 128
NC = D // C
W = 128
NWIN = N // W

K = 4
NK = N // K
NWK = NWIN // K

TC_ROWS = 2048
TB = NK // TC_ROWS


WR = 64
NWR = N // WR
NWRK = NK // WR


def _gather_sc(idx_pad, table):
    mesh = plsc.VectorSubcoreMesh(core_axis_name="c", subcore_axis_name="s")

    @functools.partial(
        pl.kernel,
        out_type=jax.ShapeDtypeStruct((NK, D), jnp.float32),
        mesh=mesh,
    )
    def k(i_hbm, t_hbm, o_hbm):
        def body(i_vmem, o_vmem):
            pltpu.sync_copy(t_hbm.at[i_vmem.at[0, pl.ds(0, WR)]], o_vmem)

        pltpu.emit_pipeline(
            body,
            grid=(NWRK,),
            in_specs=[pl.BlockSpec((1, 128), lambda i: (i, 0))],
            out_specs=[pl.BlockSpec((WR, D), lambda i: (i, 0))],
            core_axis_name=("c", "s"),
            dimension_semantics=(pltpu.PARALLEL,),
        )(i_hbm, o_hbm)

    return k(idx_pad, table)


def _add_first(x2d, pe0):
    def body(x_ref, pe_ref, o_ref):
        o_ref[...] = x_ref[...] + pe_ref[...]

    return pl.pallas_call(
        body,
        out_shape=jax.ShapeDtypeStruct((N, D), jnp.float32),
        grid=(TB,),
        in_specs=[
            pl.BlockSpec((TC_ROWS, D), lambda i: (i, 0)),
            pl.BlockSpec((TC_ROWS, D), lambda i: (i, 0)),
        ],
        out_specs=pl.BlockSpec((TC_ROWS, D), lambda i: (i, 0)),
    )(x2d, pe0)


def _add_chunk(prev, x2d, pe, chunk):
    def body(prev_ref, x_ref, pe_ref, o_ref):
        o_ref[...] = x_ref[...] + pe_ref[...]

    off = chunk * TB
    return pl.pallas_call(
        body,
        out_shape=jax.ShapeDtypeStruct((N, D), jnp.float32),
        grid=(TB,),
        in_specs=[
            pl.BlockSpec(memory_space=pltpu.MemorySpace.HBM),
            pl.BlockSpec((TC_ROWS, D), lambda i: (i + off, 0)),
            pl.BlockSpec((TC_ROWS, D), lambda i: (i, 0)),
        ],
        out_specs=pl.BlockSpec((TC_ROWS, D), lambda i: (i + off, 0)),
        input_output_aliases={0: 0},
    )(prev, x2d, pe)


def kernel(x, position, pos_table):
    x2d = x.reshape(N, D)
    pos = position.reshape(NWR, WR).astype(jnp.int32)
    idx_pad = jnp.concatenate([pos, pos], axis=1).reshape(K, NWRK, 2 * WR)
    pe = [_gather_sc(idx_pad[k], pos_table) for k in range(K)]
    out = _add_first(x2d, pe[0])
    for k in range(1, K):
        out = _add_chunk(out, x2d, pe[k], k)
    return out.reshape(B, S, D)


# speedup vs baseline: 2.2490x; 1.0099x over previous
import functools

import jax
import jax.numpy as jnp
from jax.experimental import pallas as pl
from jax.experimental.pallas import tpu as pltpu
from jax.experimental.pallas import tpu_sc as plsc

B = 4
S = 8192
D = 768
N = B * S
C = 128
NC = D // C
W = 128
NWIN = N // W

K = 2
NK = N // K
NWK = NWIN // K

TC_ROWS = 2048
TB = NK // TC_ROWS


WR = 64
NWR = N // WR
NWRK = NK // WR


def _gather_sc(idx_pad, table):
    mesh = plsc.VectorSubcoreMesh(core_axis_name="c", subcore_axis_name="s")

    @functools.partial(
        pl.kernel,
        out_type=jax.ShapeDtypeStruct((NK, D), jnp.float32),
        mesh=mesh,
    )
    def k(i_hbm, t_hbm, o_hbm):
        def body(i_vmem, o_vmem):
            pltpu.sync_copy(t_hbm.at[i_vmem.at[0, pl.ds(0, WR)]], o_vmem)

        pltpu.emit_pipeline(
            body,
            grid=(NWRK,),
            in_specs=[pl.BlockSpec((1, 128), lambda i: (i, 0))],
            out_specs=[pl.BlockSpec((WR, D), lambda i: (i, 0))],
            core_axis_name=("c", "s"),
            dimension_semantics=(pltpu.PARALLEL,),
        )(i_hbm, o_hbm)

    return k(idx_pad, table)


def _add_first(x2d, pe0):
    def body(x_ref, pe_ref, o_ref):
        o_ref[...] = x_ref[...] + pe_ref[...]

    return pl.pallas_call(
        body,
        out_shape=jax.ShapeDtypeStruct((N, D), jnp.float32),
        grid=(TB,),
        in_specs=[
            pl.BlockSpec((TC_ROWS, D), lambda i: (i, 0)),
            pl.BlockSpec((TC_ROWS, D), lambda i: (i, 0)),
        ],
        out_specs=pl.BlockSpec((TC_ROWS, D), lambda i: (i, 0)),
    )(x2d, pe0)


def _add_chunk(prev, x2d, pe, chunk):
    def body(prev_ref, x_ref, pe_ref, o_ref):
        o_ref[...] = x_ref[...] + pe_ref[...]

    off = chunk * TB
    return pl.pallas_call(
        body,
        out_shape=jax.ShapeDtypeStruct((N, D), jnp.float32),
        grid=(TB,),
        in_specs=[
            pl.BlockSpec(memory_space=pltpu.MemorySpace.HBM),
            pl.BlockSpec((TC_ROWS, D), lambda i: (i + off, 0)),
            pl.BlockSpec((TC_ROWS, D), lambda i: (i, 0)),
        ],
        out_specs=pl.BlockSpec((TC_ROWS, D), lambda i: (i + off, 0)),
        input_output_aliases={0: 0},
    )(prev, x2d, pe)


def kernel(x, position, pos_table):
    x2d = x.reshape(N, D)
    pos = position.reshape(NWR, WR).astype(jnp.int32)
    idx_pad = jnp.concatenate([pos, pos], axis=1).reshape(K, NWRK, 2 * WR)
    pe = [_gather_sc(idx_pad[k], pos_table) for k in range(K)]
    out = _add_first(x2d, pe[0])
    for k in range(1, K):
        out = _add_chunk(out, x2d, pe[k], k)
    return out.reshape(B, S, D)
